# baseline (speedup 1.0000x reference)
import functools

import jax
import jax.numpy as jnp
from jax import lax
from jax.experimental import pallas as pl
from jax.experimental.pallas import tpu as pltpu
from jax.experimental.pallas import tpu_sc as plsc

NC, NS, L = 2, 16, 16
CHUNK = 128
HD = 128


def _sc_mesh():
    return plsc.VectorSubcoreMesh(core_axis_name="c", subcore_axis_name="s")


def _make_agg_kernel(n_pad, k_chunks):
    rps = n_pad // NS
    kb = k_chunks

    @functools.partial(
        pl.kernel,
        out_type=jax.ShapeDtypeStruct((NC, n_pad, HD), jnp.float32),
        mesh=_sc_mesh(),
        scratch_types=[
            pltpu.VMEM((kb, CHUNK), jnp.int32),
            pltpu.VMEM((kb, CHUNK), jnp.int32),
            pltpu.VMEM((CHUNK, HD), jnp.float32),
            pltpu.VMEM_SHARED((n_pad, HD), jnp.float32),
            pltpu.SemaphoreType.DMA,
        ],
    )
    def agg_kernel(g_hbm, src_hbm, dst_hbm, zeros_hbm, out_hbm,
                   src_v, dst_v, rows_v, acc, sem):
        c = lax.axis_index("c")
        s = lax.axis_index("s")
        pltpu.sync_copy(src_hbm.at[s], src_v)
        pltpu.sync_copy(dst_hbm.at[s], dst_v)
        pltpu.sync_copy(zeros_hbm, acc.at[pl.ds(s * rps, rps)])
        plsc.subcore_barrier()

        @pl.loop(0, kb)
        def _(j):
            pltpu.async_copy(g_hbm.at[c].at[src_v.at[j]], rows_v, sem).wait()
            pltpu.sync_copy(rows_v, acc.at[dst_v.at[j]], add=True)

        plsc.subcore_barrier()
        pltpu.sync_copy(acc.at[pl.ds(s * rps, rps)],
                        out_hbm.at[c, pl.ds(s * rps, rps)])

    return agg_kernel


def _dis_from_deg(deg_ref):
    degsum = deg_ref[0] + 1.0
    return lax.rsqrt(degsum[:, 0:1])


def _mm1_body(x_ref, deg_ref, w_ref, g_ref):
    dis = _dis_from_deg(deg_ref)
    h = jnp.dot(x_ref[...], w_ref[...], preferred_element_type=jnp.float32)
    g = h * dis
    g_ref[0] = g[:, :HD]
    g_ref[1] = g[:, HD:]


def _mid_body(acc_ref, g_ref, deg_ref, b_ref, w_ref, gout_ref):
    dis = _dis_from_deg(deg_ref)
    m = jnp.concatenate([acc_ref[0] + g_ref[0], acc_ref[1] + g_ref[1]], axis=1)
    h = jnp.maximum(m * dis + b_ref[...], 0.0)
    h2 = jnp.dot(h, w_ref[...], preferred_element_type=jnp.float32)
    g2 = h2 * dis
    gout_ref[0] = g2[:, :HD]
    gout_ref[1] = g2[:, HD:]


def _head_body(acc_ref, g_ref, deg_ref, b_ref, wf_ref, bf_ref, wo_ref, bo_ref,
               y_ref):
    dis = _dis_from_deg(deg_ref)
    m = jnp.concatenate([acc_ref[0] + g_ref[0], acc_ref[1] + g_ref[1]], axis=1)
    h = jnp.maximum(m * dis + b_ref[...], 0.0)
    f = jnp.maximum(
        jnp.dot(h, wf_ref[...], preferred_element_type=jnp.float32)
        + bf_ref[...], 0.0)
    y_ref[...] = (jnp.dot(f, wo_ref[...], preferred_element_type=jnp.float32)
                  + bo_ref[...])


def kernel(x, edge_index, W_g1, b_g1, W_g2, b_g2, W_f1, b_f1, W_out, b_out):
    n, d = x.shape
    e = edge_index.shape[1]
    n_pad = -(-(n + 1) // 1024) * 1024
    k_chunks = -(-e // (NS * CHUNK))
    e_pad = NS * k_chunks * CHUNK

    src = jnp.concatenate(
        [edge_index[0], jnp.full((e_pad - e,), n, jnp.int32)]).reshape(
            NS, k_chunks, CHUNK)
    dst = jnp.concatenate(
        [edge_index[1], jnp.full((e_pad - e,), n, jnp.int32)]).reshape(
            NS, k_chunks, CHUNK)
    x_p = jnp.pad(x, ((0, n_pad - n), (0, 0)))

    rps = n_pad // NS
    zeros_hd = jnp.zeros((rps, HD), jnp.float32)

    agg = _make_agg_kernel(n_pad, k_chunks)
    ones_hd = jnp.ones((NC, n_pad, HD), jnp.float32)
    zero_idx = jnp.zeros_like(dst)
    deg = agg(ones_hd, zero_idx, dst, zeros_hd)

    rb = 1024
    grid = (n_pad // rb,)
    f32 = jnp.float32

    half_spec = pl.BlockSpec((NC, rb, HD), lambda i: (0, i, 0))
    deg_spec = pl.BlockSpec((NC, rb, HD), lambda i: (0, i, 0))
    full_spec = lambda r, c: pl.BlockSpec((r, c), lambda i: (0, 0))

    g1 = pl.pallas_call(
        _mm1_body,
        grid=grid,
        in_specs=[
            pl.BlockSpec((rb, d), lambda i: (i, 0)),
            deg_spec,
            full_spec(d, d),
        ],
        out_specs=half_spec,
        out_shape=jax.ShapeDtypeStruct((NC, n_pad, HD), f32),
    )(x_p, deg, W_g1)

    acc1 = agg(g1, src, dst, zeros_hd)

    g2 = pl.pallas_call(
        _mid_body,
        grid=grid,
        in_specs=[
            half_spec,
            half_spec,
            deg_spec,
            full_spec(1, d),
            full_spec(d, d),
        ],
        out_specs=half_spec,
        out_shape=jax.ShapeDtypeStruct((NC, n_pad, HD), f32),
    )(acc1, g1, deg, b_g1.reshape(1, d), W_g2)

    acc2 = agg(g2, src, dst, zeros_hd)

    df = W_f1.shape[1]
    y = pl.pallas_call(
        _head_body,
        grid=grid,
        in_specs=[
            half_spec,
            half_spec,
            deg_spec,
            full_spec(1, d),
            full_spec(d, df),
            full_spec(1, df),
            full_spec(df, 1),
            full_spec(1, 1),
        ],
        out_specs=pl.BlockSpec((rb, 1), lambda i: (i, 0)),
        out_shape=jax.ShapeDtypeStruct((n_pad, 1), f32),
    )(acc2, g2, deg, b_g2.reshape(1, d), W_f1, b_f1.reshape(1, df),
      W_out, b_out.reshape(1, 1))

    return y[:n]


# speedup vs baseline: 8.1058x; 8.1058x over previous
import functools

import jax
import jax.numpy as jnp
from jax import lax
from jax.experimental import pallas as pl
from jax.experimental.pallas import tpu as pltpu
from jax.experimental.pallas import tpu_sc as plsc

NC, NS, L = 2, 16, 16
CHUNK = 128
HD = 128


def _sc_mesh():
    return plsc.VectorSubcoreMesh(core_axis_name="c", subcore_axis_name="s")


def _make_agg_kernel(n_pad, k_chunks):
    rps = n_pad // NS
    kb = k_chunks

    @functools.partial(
        pl.kernel,
        out_type=jax.ShapeDtypeStruct((NC, n_pad, HD), jnp.float32),
        mesh=_sc_mesh(),
        scratch_types=[
            pltpu.VMEM((kb, CHUNK), jnp.int32),
            pltpu.VMEM((kb, CHUNK), jnp.int32),
            pltpu.VMEM((CHUNK, HD), jnp.float32),
            pltpu.VMEM_SHARED((n_pad, HD), jnp.float32),
            pltpu.SemaphoreType.DMA,
        ],
    )
    def agg_kernel(g_hbm, src_hbm, dst_hbm, zeros_hbm, out_hbm,
                   src_v, dst_v, rows_v, acc, sem):
        c = lax.axis_index("c")
        s = lax.axis_index("s")
        pltpu.sync_copy(src_hbm.at[s], src_v)
        pltpu.sync_copy(dst_hbm.at[s], dst_v)
        pltpu.sync_copy(zeros_hbm, acc.at[pl.ds(s * rps, rps)])
        plsc.subcore_barrier()

        @pl.loop(0, kb)
        def _(j):
            pltpu.async_copy(g_hbm.at[c].at[src_v.at[j]], rows_v, sem).wait()
            pltpu.sync_copy(rows_v, acc.at[dst_v.at[j]], add=True)

        plsc.subcore_barrier()
        pltpu.sync_copy(acc.at[pl.ds(s * rps, rps)],
                        out_hbm.at[c, pl.ds(s * rps, rps)])

    return agg_kernel


def _dis_from_deg(deg_ref):
    degsum = deg_ref[0] + 1.0
    return lax.rsqrt(degsum[:, 0:1])


def _mm1_body(x_ref, deg_ref, w_ref, g_ref):
    dis = _dis_from_deg(deg_ref)
    h = jnp.dot(x_ref[...], w_ref[...], preferred_element_type=jnp.float32)
    g = h * dis
    g_ref[0] = g[:, :HD]
    g_ref[1] = g[:, HD:]


def _mid_body(acc_ref, g_ref, deg_ref, b_ref, w_ref, gout_ref):
    dis = _dis_from_deg(deg_ref)
    m = jnp.concatenate([acc_ref[0] + g_ref[0], acc_ref[1] + g_ref[1]], axis=1)
    h = jnp.maximum(m * dis + b_ref[...], 0.0)
    h2 = jnp.dot(h, w_ref[...], preferred_element_type=jnp.float32)
    g2 = h2 * dis
    gout_ref[0] = g2[:, :HD]
    gout_ref[1] = g2[:, HD:]


def _head_body(acc_ref, g_ref, deg_ref, b_ref, wf_ref, bf_ref, wo_ref, bo_ref,
               y_ref):
    dis = _dis_from_deg(deg_ref)
    m = jnp.concatenate([acc_ref[0] + g_ref[0], acc_ref[1] + g_ref[1]], axis=1)
    h = jnp.maximum(m * dis + b_ref[...], 0.0)
    f = jnp.maximum(
        jnp.dot(h, wf_ref[...], preferred_element_type=jnp.float32)
        + bf_ref[...], 0.0)
    y_ref[...] = (jnp.dot(f, wo_ref[...], preferred_element_type=jnp.float32)
                  + bo_ref[...])


def kernel(x, edge_index, W_g1, b_g1, W_g2, b_g2, W_f1, b_f1, W_out, b_out):
    n, d = x.shape
    e = edge_index.shape[1]
    n_pad = -(-(n + 1) // 1024) * 1024
    k_chunks = -(-e // (NS * CHUNK))
    e_pad = NS * k_chunks * CHUNK

    src = jnp.concatenate(
        [edge_index[0], jnp.full((e_pad - e,), n, jnp.int32)]).reshape(
            NS, k_chunks, CHUNK)
    dst = jnp.concatenate(
        [edge_index[1], jnp.full((e_pad - e,), n, jnp.int32)]).reshape(
            NS, k_chunks, CHUNK)
    x_p = jnp.pad(x, ((0, n_pad - n), (0, 0)))

    rps = n_pad // NS
    zeros_hd = jnp.zeros((rps, HD), jnp.float32)

    agg = _make_agg_kernel(n_pad, k_chunks)
    ones_hd = jnp.ones((NC, n_pad, HD), jnp.float32)
    deg = agg(ones_hd, dst, dst, zeros_hd)

    rb = 1024
    grid = (n_pad // rb,)
    f32 = jnp.float32

    half_spec = pl.BlockSpec((NC, rb, HD), lambda i: (0, i, 0))
    deg_spec = pl.BlockSpec((NC, rb, HD), lambda i: (0, i, 0))
    full_spec = lambda r, c: pl.BlockSpec((r, c), lambda i: (0, 0))

    g1 = pl.pallas_call(
        _mm1_body,
        grid=grid,
        in_specs=[
            pl.BlockSpec((rb, d), lambda i: (i, 0)),
            deg_spec,
            full_spec(d, d),
        ],
        out_specs=half_spec,
        out_shape=jax.ShapeDtypeStruct((NC, n_pad, HD), f32),
    )(x_p, deg, W_g1)

    acc1 = agg(g1, src, dst, zeros_hd)

    g2 = pl.pallas_call(
        _mid_body,
        grid=grid,
        in_specs=[
            half_spec,
            half_spec,
            deg_spec,
            full_spec(1, d),
            full_spec(d, d),
        ],
        out_specs=half_spec,
        out_shape=jax.ShapeDtypeStruct((NC, n_pad, HD), f32),
    )(acc1, g1, deg, b_g1.reshape(1, d), W_g2)

    acc2 = agg(g2, src, dst, zeros_hd)

    df = W_f1.shape[1]
    y = pl.pallas_call(
        _head_body,
        grid=grid,
        in_specs=[
            half_spec,
            half_spec,
            deg_spec,
            full_spec(1, d),
            full_spec(d, df),
            full_spec(1, df),
            full_spec(df, 1),
            full_spec(1, 1),
        ],
        out_specs=pl.BlockSpec((rb, 1), lambda i: (i, 0)),
        out_shape=jax.ShapeDtypeStruct((n_pad, 1), f32),
    )(acc2, g2, deg, b_g2.reshape(1, d), W_f1, b_f1.reshape(1, df),
      W_out, b_out.reshape(1, 1))

    return y[:n]


# speedup vs baseline: 9.6328x; 1.1884x over previous
import functools

import jax
import jax.numpy as jnp
from jax import lax
from jax.experimental import pallas as pl
from jax.experimental.pallas import tpu as pltpu
from jax.experimental.pallas import tpu_sc as plsc

NC, NS, L = 2, 16, 16
CHUNK = 128
HD = 128


def _sc_mesh():
    return plsc.VectorSubcoreMesh(core_axis_name="c", subcore_axis_name="s")


def _make_deg_kernel(n_pad, k_chunks):
    rps = n_pad // NS

    @functools.partial(
        pl.kernel,
        out_type=jax.ShapeDtypeStruct((NC, n_pad, HD), jnp.float32),
        mesh=_sc_mesh(),
        scratch_types=[
            pltpu.VMEM((k_chunks, CHUNK), jnp.int32),
            pltpu.VMEM((CHUNK, HD), jnp.float32),
            pltpu.VMEM_SHARED((n_pad, HD), jnp.float32),
            pltpu.SemaphoreType.DMA,
        ],
    )
    def deg_kernel(dst_hbm, ones_hbm, zeros_hbm, out_hbm,
                   dst_v, ones_v, acc, sem):
        c = lax.axis_index("c")
        s = lax.axis_index("s")
        pltpu.sync_copy(dst_hbm.at[s], dst_v)
        pltpu.sync_copy(ones_hbm, ones_v)
        pltpu.sync_copy(zeros_hbm, acc.at[pl.ds(s * rps, rps)])
        plsc.subcore_barrier()

        @pl.loop(0, k_chunks)
        def _(j):
            pltpu.async_copy(ones_v, acc.at[dst_v.at[j]], sem, add=True).wait()

        plsc.subcore_barrier()
        pltpu.sync_copy(acc.at[pl.ds(s * rps, rps)],
                        out_hbm.at[c, pl.ds(s * rps, rps)])

    return deg_kernel


def _make_agg_kernel(n_pad, k_chunks):
    rps = n_pad // NS
    kb = k_chunks

    @functools.partial(
        pl.kernel,
        out_type=jax.ShapeDtypeStruct((NC, n_pad, HD), jnp.float32),
        mesh=_sc_mesh(),
        scratch_types=[
            pltpu.VMEM((kb, CHUNK), jnp.int32),
            pltpu.VMEM((kb, CHUNK), jnp.int32),
            pltpu.VMEM((CHUNK, HD), jnp.float32),
            pltpu.VMEM_SHARED((n_pad, HD), jnp.float32),
            pltpu.SemaphoreType.DMA,
        ],
    )
    def agg_kernel(g_hbm, src_hbm, dst_hbm, zeros_hbm, out_hbm,
                   src_v, dst_v, rows_v, acc, sem):
        c = lax.axis_index("c")
        s = lax.axis_index("s")
        pltpu.sync_copy(src_hbm.at[s], src_v)
        pltpu.sync_copy(dst_hbm.at[s], dst_v)
        pltpu.sync_copy(zeros_hbm, acc.at[pl.ds(s * rps, rps)])
        plsc.subcore_barrier()

        @pl.loop(0, kb)
        def _(j):
            pltpu.async_copy(g_hbm.at[c].at[src_v.at[j]], rows_v, sem).wait()
            pltpu.sync_copy(rows_v, acc.at[dst_v.at[j]], add=True)

        plsc.subcore_barrier()
        pltpu.sync_copy(acc.at[pl.ds(s * rps, rps)],
                        out_hbm.at[c, pl.ds(s * rps, rps)])

    return agg_kernel


def _dis_from_deg(deg_ref):
    degsum = deg_ref[0] + 1.0
    return lax.rsqrt(degsum[:, 0:1])


def _mm1_body(x_ref, deg_ref, w_ref, g_ref):
    dis = _dis_from_deg(deg_ref)
    h = jnp.dot(x_ref[...], w_ref[...], preferred_element_type=jnp.float32)
    g = h * dis
    g_ref[0] = g[:, :HD]
    g_ref[1] = g[:, HD:]


def _mid_body(acc_ref, g_ref, deg_ref, b_ref, w_ref, gout_ref):
    dis = _dis_from_deg(deg_ref)
    m = jnp.concatenate([acc_ref[0] + g_ref[0], acc_ref[1] + g_ref[1]], axis=1)
    h = jnp.maximum(m * dis + b_ref[...], 0.0)
    h2 = jnp.dot(h, w_ref[...], preferred_element_type=jnp.float32)
    g2 = h2 * dis
    gout_ref[0] = g2[:, :HD]
    gout_ref[1] = g2[:, HD:]


def _head_body(acc_ref, g_ref, deg_ref, b_ref, wf_ref, bf_ref, wo_ref, bo_ref,
               y_ref):
    dis = _dis_from_deg(deg_ref)
    m = jnp.concatenate([acc_ref[0] + g_ref[0], acc_ref[1] + g_ref[1]], axis=1)
    h = jnp.maximum(m * dis + b_ref[...], 0.0)
    f = jnp.maximum(
        jnp.dot(h, wf_ref[...], preferred_element_type=jnp.float32)
        + bf_ref[...], 0.0)
    y_ref[...] = (jnp.dot(f, wo_ref[...], preferred_element_type=jnp.float32)
                  + bo_ref[...])


def kernel(x, edge_index, W_g1, b_g1, W_g2, b_g2, W_f1, b_f1, W_out, b_out):
    n, d = x.shape
    e = edge_index.shape[1]
    n_pad = -(-(n + 1) // 1024) * 1024
    k_chunks = -(-e // (NS * CHUNK))
    e_pad = NS * k_chunks * CHUNK

    src = jnp.concatenate(
        [edge_index[0], jnp.full((e_pad - e,), n, jnp.int32)]).reshape(
            NS, k_chunks, CHUNK)
    dst = jnp.concatenate(
        [edge_index[1], jnp.full((e_pad - e,), n, jnp.int32)]).reshape(
            NS, k_chunks, CHUNK)
    x_p = jnp.pad(x, ((0, n_pad - n), (0, 0)))

    rps = n_pad // NS
    zeros_hd = jnp.zeros((rps, HD), jnp.float32)

    agg = _make_agg_kernel(n_pad, k_chunks)
    ones_rows = jnp.ones((CHUNK, HD), jnp.float32)
    deg = _make_deg_kernel(n_pad, k_chunks)(dst, ones_rows, zeros_hd)

    rb = 1024
    grid = (n_pad // rb,)
    f32 = jnp.float32

    half_spec = pl.BlockSpec((NC, rb, HD), lambda i: (0, i, 0))
    deg_spec = pl.BlockSpec((NC, rb, HD), lambda i: (0, i, 0))
    full_spec = lambda r, c: pl.BlockSpec((r, c), lambda i: (0, 0))

    g1 = pl.pallas_call(
        _mm1_body,
        grid=grid,
        in_specs=[
            pl.BlockSpec((rb, d), lambda i: (i, 0)),
            deg_spec,
            full_spec(d, d),
        ],
        out_specs=half_spec,
        out_shape=jax.ShapeDtypeStruct((NC, n_pad, HD), f32),
    )(x_p, deg, W_g1)

    acc1 = agg(g1, src, dst, zeros_hd)

    g2 = pl.pallas_call(
        _mid_body,
        grid=grid,
        in_specs=[
            half_spec,
            half_spec,
            deg_spec,
            full_spec(1, d),
            full_spec(d, d),
        ],
        out_specs=half_spec,
        out_shape=jax.ShapeDtypeStruct((NC, n_pad, HD), f32),
    )(acc1, g1, deg, b_g1.reshape(1, d), W_g2)

    acc2 = agg(g2, src, dst, zeros_hd)

    df = W_f1.shape[1]
    y = pl.pallas_call(
        _head_body,
        grid=grid,
        in_specs=[
            half_spec,
            half_spec,
            deg_spec,
            full_spec(1, d),
            full_spec(d, df),
            full_spec(1, df),
            full_spec(df, 1),
            full_spec(1, 1),
        ],
        out_specs=pl.BlockSpec((rb, 1), lambda i: (i, 0)),
        out_shape=jax.ShapeDtypeStruct((n_pad, 1), f32),
    )(acc2, g2, deg, b_g2.reshape(1, d), W_f1, b_f1.reshape(1, df),
      W_out, b_out.reshape(1, 1))

    return y[:n]


# speedup vs baseline: 10.0050x; 1.0386x over previous
import functools

import jax
import jax.numpy as jnp
from jax import lax
from jax.experimental import pallas as pl
from jax.experimental.pallas import tpu as pltpu
from jax.experimental.pallas import tpu_sc as plsc

NC, NS, L = 2, 16, 16
CHUNK = 128
HD = 128


def _sc_mesh():
    return plsc.VectorSubcoreMesh(core_axis_name="c", subcore_axis_name="s")


def _make_deg_kernel(n_pad, k_chunks):
    rps = n_pad // NS

    @functools.partial(
        pl.kernel,
        out_type=jax.ShapeDtypeStruct((NC, n_pad, HD), jnp.float32),
        mesh=_sc_mesh(),
        scratch_types=[
            pltpu.VMEM((k_chunks, CHUNK), jnp.int32),
            pltpu.VMEM((CHUNK, HD), jnp.float32),
            pltpu.VMEM_SHARED((n_pad, HD), jnp.float32),
            pltpu.SemaphoreType.DMA,
        ],
    )
    def deg_kernel(dst_hbm, ones_hbm, zeros_hbm, out_hbm,
                   dst_v, ones_v, acc, sem):
        c = lax.axis_index("c")
        s = lax.axis_index("s")
        pltpu.sync_copy(dst_hbm.at[s], dst_v)
        pltpu.sync_copy(ones_hbm, ones_v)
        pltpu.sync_copy(zeros_hbm, acc.at[pl.ds(s * rps, rps)])
        plsc.subcore_barrier()

        @pl.loop(0, k_chunks)
        def _(j):
            pltpu.async_copy(ones_v, acc.at[dst_v.at[j]], sem, add=True).wait()

        plsc.subcore_barrier()
        pltpu.sync_copy(acc.at[pl.ds(s * rps, rps)],
                        out_hbm.at[c, pl.ds(s * rps, rps)])

    return deg_kernel


def _make_agg_kernel(n_pad, k_chunks):
    rps = n_pad // NS
    kb = k_chunks

    @functools.partial(
        pl.kernel,
        out_type=jax.ShapeDtypeStruct((NC, n_pad, HD), jnp.float32),
        mesh=_sc_mesh(),
        scratch_types=[
            pltpu.VMEM((kb, CHUNK // 2), jnp.int32),
            pltpu.VMEM((kb, CHUNK // 2), jnp.int32),
            pltpu.VMEM((kb, CHUNK), jnp.int32),
            pltpu.VMEM((CHUNK, HD), jnp.float32),
            pltpu.VMEM_SHARED((n_pad, HD), jnp.float32),
            pltpu.SemaphoreType.DMA,
            pltpu.SemaphoreType.DMA,
            pltpu.SemaphoreType.DMA,
        ],
    )
    def agg_kernel(g_hbm, srcu_hbm, srcl_hbm, dst_hbm, zeros_hbm, out_hbm,
                   srcu_v, srcl_v, dst_v, rows_v, acc, semu, seml, sems):
        c = lax.axis_index("c")
        s = lax.axis_index("s")
        pltpu.sync_copy(srcu_hbm.at[s], srcu_v)
        pltpu.sync_copy(srcl_hbm.at[s], srcl_v)
        pltpu.sync_copy(dst_hbm.at[s], dst_v)
        pltpu.sync_copy(zeros_hbm, acc.at[pl.ds(s * rps, rps)])
        plsc.subcore_barrier()

        up = rows_v.at[pl.ds(0, CHUNK // 2)]
        lo = rows_v.at[pl.ds(CHUNK // 2, CHUNK // 2)]

        @pl.loop(0, kb)
        def _(j):
            pltpu.async_copy(g_hbm.at[c].at[srcu_v.at[j]], up, semu)
            pltpu.async_copy(g_hbm.at[c].at[srcl_v.at[j]], lo, seml)
            pltpu.make_async_copy(g_hbm.at[c].at[srcu_v.at[j]],
                                  up, semu).wait()
            pltpu.make_async_copy(g_hbm.at[c].at[srcl_v.at[j]],
                                  lo, seml).wait()
            pltpu.async_copy(rows_v, acc.at[dst_v.at[j]], sems,
                             add=True).wait()

        plsc.subcore_barrier()
        pltpu.sync_copy(acc.at[pl.ds(s * rps, rps)],
                        out_hbm.at[c, pl.ds(s * rps, rps)])

    return agg_kernel


def _dis_from_deg(deg_ref):
    degsum = deg_ref[0] + 1.0
    return lax.rsqrt(degsum[:, 0:1])


def _mm1_body(x_ref, deg_ref, w_ref, g_ref):
    dis = _dis_from_deg(deg_ref)
    h = jnp.dot(x_ref[...], w_ref[...], preferred_element_type=jnp.float32)
    g = h * dis
    g_ref[0] = g[:, :HD]
    g_ref[1] = g[:, HD:]


def _mid_body(acc_ref, g_ref, deg_ref, b_ref, w_ref, gout_ref):
    dis = _dis_from_deg(deg_ref)
    m = jnp.concatenate([acc_ref[0] + g_ref[0], acc_ref[1] + g_ref[1]], axis=1)
    h = jnp.maximum(m * dis + b_ref[...], 0.0)
    h2 = jnp.dot(h, w_ref[...], preferred_element_type=jnp.float32)
    g2 = h2 * dis
    gout_ref[0] = g2[:, :HD]
    gout_ref[1] = g2[:, HD:]


def _head_body(acc_ref, g_ref, deg_ref, b_ref, wf_ref, bf_ref, wo_ref, bo_ref,
               y_ref):
    dis = _dis_from_deg(deg_ref)
    m = jnp.concatenate([acc_ref[0] + g_ref[0], acc_ref[1] + g_ref[1]], axis=1)
    h = jnp.maximum(m * dis + b_ref[...], 0.0)
    f = jnp.maximum(
        jnp.dot(h, wf_ref[...], preferred_element_type=jnp.float32)
        + bf_ref[...], 0.0)
    y_ref[...] = (jnp.dot(f, wo_ref[...], preferred_element_type=jnp.float32)
                  + bo_ref[...])


def kernel(x, edge_index, W_g1, b_g1, W_g2, b_g2, W_f1, b_f1, W_out, b_out):
    n, d = x.shape
    e = edge_index.shape[1]
    n_pad = -(-(n + 1) // 1024) * 1024
    k_chunks = -(-e // (NS * CHUNK))
    e_pad = NS * k_chunks * CHUNK

    src = jnp.concatenate(
        [edge_index[0], jnp.full((e_pad - e,), n, jnp.int32)]).reshape(
            NS, k_chunks, CHUNK)
    dst = jnp.concatenate(
        [edge_index[1], jnp.full((e_pad - e,), n, jnp.int32)]).reshape(
            NS, k_chunks, CHUNK)
    x_p = jnp.pad(x, ((0, n_pad - n), (0, 0)))

    rps = n_pad // NS
    zeros_hd = jnp.zeros((rps, HD), jnp.float32)

    agg = _make_agg_kernel(n_pad, k_chunks)
    ones_rows = jnp.ones((CHUNK, HD), jnp.float32)
    deg = _make_deg_kernel(n_pad, k_chunks)(dst, ones_rows, zeros_hd)

    rb = 1024
    grid = (n_pad // rb,)
    f32 = jnp.float32

    half_spec = pl.BlockSpec((NC, rb, HD), lambda i: (0, i, 0))
    deg_spec = pl.BlockSpec((NC, rb, HD), lambda i: (0, i, 0))
    full_spec = lambda r, c: pl.BlockSpec((r, c), lambda i: (0, 0))

    g1 = pl.pallas_call(
        _mm1_body,
        grid=grid,
        in_specs=[
            pl.BlockSpec((rb, d), lambda i: (i, 0)),
            deg_spec,
            full_spec(d, d),
        ],
        out_specs=half_spec,
        out_shape=jax.ShapeDtypeStruct((NC, n_pad, HD), f32),
    )(x_p, deg, W_g1)

    srcu = src[:, :, :CHUNK // 2]
    srcl = src[:, :, CHUNK // 2:]
    acc1 = agg(g1, srcu, srcl, dst, zeros_hd)

    g2 = pl.pallas_call(
        _mid_body,
        grid=grid,
        in_specs=[
            half_spec,
            half_spec,
            deg_spec,
            full_spec(1, d),
            full_spec(d, d),
        ],
        out_specs=half_spec,
        out_shape=jax.ShapeDtypeStruct((NC, n_pad, HD), f32),
    )(acc1, g1, deg, b_g1.reshape(1, d), W_g2)

    acc2 = agg(g2, srcu, srcl, dst, zeros_hd)

    df = W_f1.shape[1]
    y = pl.pallas_call(
        _head_body,
        grid=grid,
        in_specs=[
            half_spec,
            half_spec,
            deg_spec,
            full_spec(1, d),
            full_spec(d, df),
            full_spec(1, df),
            full_spec(df, 1),
            full_spec(1, 1),
        ],
        out_specs=pl.BlockSpec((rb, 1), lambda i: (i, 0)),
        out_shape=jax.ShapeDtypeStruct((n_pad, 1), f32),
    )(acc2, g2, deg, b_g2.reshape(1, d), W_f1, b_f1.reshape(1, df),
      W_out, b_out.reshape(1, 1))

    return y[:n]


# speedup vs baseline: 10.0109x; 1.0006x over previous
import functools

import jax
import jax.numpy as jnp
from jax import lax
from jax.experimental import pallas as pl
from jax.experimental.pallas import tpu as pltpu
from jax.experimental.pallas import tpu_sc as plsc

NC, NS, L = 2, 16, 16
CHUNK = 128
HD = 128


def _sc_mesh():
    return plsc.VectorSubcoreMesh(core_axis_name="c", subcore_axis_name="s")


def _make_deg_kernel(n_pad, k_chunks):
    rps = n_pad // NS

    @functools.partial(
        pl.kernel,
        out_type=jax.ShapeDtypeStruct((NC, n_pad, HD), jnp.float32),
        mesh=_sc_mesh(),
        scratch_types=[
            pltpu.VMEM((k_chunks, CHUNK), jnp.int32),
            pltpu.VMEM((CHUNK, HD), jnp.float32),
            pltpu.VMEM_SHARED((n_pad, HD), jnp.float32),
            pltpu.SemaphoreType.DMA,
        ],
    )
    def deg_kernel(dst_hbm, ones_hbm, zeros_hbm, out_hbm,
                   dst_v, ones_v, acc, sem):
        c = lax.axis_index("c")
        s = lax.axis_index("s")
        pltpu.sync_copy(dst_hbm.at[s], dst_v)
        pltpu.sync_copy(ones_hbm, ones_v)
        pltpu.sync_copy(zeros_hbm, acc.at[pl.ds(s * rps, rps)])
        plsc.subcore_barrier()

        @pl.loop(0, k_chunks)
        def _(j):
            pltpu.async_copy(ones_v, acc.at[dst_v.at[j]], sem, add=True)

        @pl.loop(0, k_chunks)
        def _(j):
            pltpu.make_async_copy(ones_v, acc.at[dst_v.at[j]], sem).wait()

        plsc.subcore_barrier()
        pltpu.sync_copy(acc.at[pl.ds(s * rps, rps)],
                        out_hbm.at[c, pl.ds(s * rps, rps)])

    return deg_kernel


def _make_agg_kernel(n_pad, k_chunks):
    rps = n_pad // NS
    kb = k_chunks

    @functools.partial(
        pl.kernel,
        out_type=jax.ShapeDtypeStruct((NC, n_pad, HD), jnp.float32),
        mesh=_sc_mesh(),
        scratch_types=[
            pltpu.VMEM((kb, CHUNK // 2), jnp.int32),
            pltpu.VMEM((kb, CHUNK // 2), jnp.int32),
            pltpu.VMEM((kb, CHUNK), jnp.int32),
            pltpu.VMEM((CHUNK, HD), jnp.float32),
            pltpu.VMEM_SHARED((n_pad, HD), jnp.float32),
            pltpu.SemaphoreType.DMA,
            pltpu.SemaphoreType.DMA,
            pltpu.SemaphoreType.DMA,
        ],
    )
    def agg_kernel(g_hbm, srcu_hbm, srcl_hbm, dst_hbm, zeros_hbm, out_hbm,
                   srcu_v, srcl_v, dst_v, rows_v, acc, semu, seml, sems):
        c = lax.axis_index("c")
        s = lax.axis_index("s")
        pltpu.sync_copy(srcu_hbm.at[s], srcu_v)
        pltpu.sync_copy(srcl_hbm.at[s], srcl_v)
        pltpu.sync_copy(dst_hbm.at[s], dst_v)
        pltpu.sync_copy(zeros_hbm, acc.at[pl.ds(s * rps, rps)])
        plsc.subcore_barrier()

        up = rows_v.at[pl.ds(0, CHUNK // 2)]
        lo = rows_v.at[pl.ds(CHUNK // 2, CHUNK // 2)]

        @pl.loop(0, kb)
        def _(j):
            pltpu.async_copy(g_hbm.at[c].at[srcu_v.at[j]], up, semu)
            pltpu.async_copy(g_hbm.at[c].at[srcl_v.at[j]], lo, seml)
            pltpu.make_async_copy(g_hbm.at[c].at[srcu_v.at[j]],
                                  up, semu).wait()
            pltpu.make_async_copy(g_hbm.at[c].at[srcl_v.at[j]],
                                  lo, seml).wait()
            pltpu.async_copy(rows_v, acc.at[dst_v.at[j]], sems,
                             add=True).wait()

        plsc.subcore_barrier()
        pltpu.sync_copy(acc.at[pl.ds(s * rps, rps)],
                        out_hbm.at[c, pl.ds(s * rps, rps)])

    return agg_kernel


def _dis_from_deg(deg_ref):
    degsum = deg_ref[0] + 1.0
    return lax.rsqrt(degsum[:, 0:1])


def _mm1_body(x_ref, deg_ref, w_ref, g_ref):
    dis = _dis_from_deg(deg_ref)
    h = jnp.dot(x_ref[...], w_ref[...], preferred_element_type=jnp.float32)
    g = h * dis
    g_ref[0] = g[:, :HD]
    g_ref[1] = g[:, HD:]


def _mid_body(acc_ref, g_ref, deg_ref, b_ref, w_ref, gout_ref):
    dis = _dis_from_deg(deg_ref)
    m = jnp.concatenate([acc_ref[0] + g_ref[0], acc_ref[1] + g_ref[1]], axis=1)
    h = jnp.maximum(m * dis + b_ref[...], 0.0)
    h2 = jnp.dot(h, w_ref[...], preferred_element_type=jnp.float32)
    g2 = h2 * dis
    gout_ref[0] = g2[:, :HD]
    gout_ref[1] = g2[:, HD:]


def _head_body(acc_ref, g_ref, deg_ref, b_ref, wf_ref, bf_ref, wo_ref, bo_ref,
               y_ref):
    dis = _dis_from_deg(deg_ref)
    m = jnp.concatenate([acc_ref[0] + g_ref[0], acc_ref[1] + g_ref[1]], axis=1)
    h = jnp.maximum(m * dis + b_ref[...], 0.0)
    f = jnp.maximum(
        jnp.dot(h, wf_ref[...], preferred_element_type=jnp.float32)
        + bf_ref[...], 0.0)
    y_ref[...] = (jnp.dot(f, wo_ref[...], preferred_element_type=jnp.float32)
                  + bo_ref[...])


def kernel(x, edge_index, W_g1, b_g1, W_g2, b_g2, W_f1, b_f1, W_out, b_out):
    n, d = x.shape
    e = edge_index.shape[1]
    n_pad = -(-(n + 1) // 1024) * 1024
    k_chunks = -(-e // (NS * CHUNK))
    e_pad = NS * k_chunks * CHUNK

    src = jnp.concatenate(
        [edge_index[0], jnp.full((e_pad - e,), n, jnp.int32)]).reshape(
            NS, k_chunks, CHUNK)
    dst = jnp.concatenate(
        [edge_index[1], jnp.full((e_pad - e,), n, jnp.int32)]).reshape(
            NS, k_chunks, CHUNK)
    x_p = jnp.pad(x, ((0, n_pad - n), (0, 0)))

    rps = n_pad // NS
    zeros_hd = jnp.zeros((rps, HD), jnp.float32)

    agg = _make_agg_kernel(n_pad, k_chunks)
    ones_rows = jnp.ones((CHUNK, HD), jnp.float32)
    deg = _make_deg_kernel(n_pad, k_chunks)(dst, ones_rows, zeros_hd)

    rb = 1024
    grid = (n_pad // rb,)
    f32 = jnp.float32

    half_spec = pl.BlockSpec((NC, rb, HD), lambda i: (0, i, 0))
    deg_spec = pl.BlockSpec((NC, rb, HD), lambda i: (0, i, 0))
    full_spec = lambda r, c: pl.BlockSpec((r, c), lambda i: (0, 0))

    g1 = pl.pallas_call(
        _mm1_body,
        grid=grid,
        in_specs=[
            pl.BlockSpec((rb, d), lambda i: (i, 0)),
            deg_spec,
            full_spec(d, d),
        ],
        out_specs=half_spec,
        out_shape=jax.ShapeDtypeStruct((NC, n_pad, HD), f32),
    )(x_p, deg, W_g1)

    srcu = src[:, :, :CHUNK // 2]
    srcl = src[:, :, CHUNK // 2:]
    acc1 = agg(g1, srcu, srcl, dst, zeros_hd)

    g2 = pl.pallas_call(
        _mid_body,
        grid=grid,
        in_specs=[
            half_spec,
            half_spec,
            deg_spec,
            full_spec(1, d),
            full_spec(d, d),
        ],
        out_specs=half_spec,
        out_shape=jax.ShapeDtypeStruct((NC, n_pad, HD), f32),
    )(acc1, g1, deg, b_g1.reshape(1, d), W_g2)

    acc2 = agg(g2, srcu, srcl, dst, zeros_hd)

    df = W_f1.shape[1]
    y = pl.pallas_call(
        _head_body,
        grid=grid,
        in_specs=[
            half_spec,
            half_spec,
            deg_spec,
            full_spec(1, d),
            full_spec(d, df),
            full_spec(1, df),
            full_spec(df, 1),
            full_spec(1, 1),
        ],
        out_specs=pl.BlockSpec((rb, 1), lambda i: (i, 0)),
        out_shape=jax.ShapeDtypeStruct((n_pad, 1), f32),
    )(acc2, g2, deg, b_g2.reshape(1, d), W_f1, b_f1.reshape(1, df),
      W_out, b_out.reshape(1, 1))

    return y[:n]
